# baseline (speedup 1.0000x reference)
import functools

import jax
import jax.numpy as jnp
from jax import lax
from jax.experimental import pallas as pl
from jax.experimental.pallas import tpu as pltpu
from jax.experimental.pallas import tpu_sc as plsc

N_TILES = 16
N_CORES = 2
NW = N_TILES * N_CORES
CHUNK = 128
GRP = 8
STEPS = 2 * GRP


def _tc_matmul_bias(x, w, b, block_rows):
    M, K = x.shape
    D = w.shape[1]

    def body(x_ref, w_ref, b_ref, o_ref):
        o_ref[...] = (
            jnp.dot(x_ref[...], w_ref[...], preferred_element_type=jnp.float32)
            + b_ref[...]
        )

    return pl.pallas_call(
        body,
        out_shape=jax.ShapeDtypeStruct((M, D), jnp.float32),
        grid=(M // block_rows,),
        in_specs=[
            pl.BlockSpec((block_rows, K), lambda i: (i, 0)),
            pl.BlockSpec((K, D), lambda i: (0, 0)),
            pl.BlockSpec((1, D), lambda i: (0, 0)),
        ],
        out_specs=pl.BlockSpec((block_rows, D), lambda i: (i, 0)),
    )(x, w, b)


def _tc_relu_add_matmul_bias(p0, p1, w, b, block_rows=512):
    M, K = p0.shape
    D = w.shape[1]

    def body(p0_ref, p1_ref, w_ref, b_ref, o_ref):
        h = jnp.maximum(p0_ref[...] + p1_ref[...], 0.0)
        o_ref[...] = (
            jnp.dot(h, w_ref[...], preferred_element_type=jnp.float32) + b_ref[...]
        )

    return pl.pallas_call(
        body,
        out_shape=jax.ShapeDtypeStruct((M, D), jnp.float32),
        grid=(M // block_rows,),
        in_specs=[
            pl.BlockSpec((block_rows, K), lambda i: (i, 0)),
            pl.BlockSpec((block_rows, K), lambda i: (i, 0)),
            pl.BlockSpec((K, D), lambda i: (0, 0)),
            pl.BlockSpec((1, D), lambda i: (0, 0)),
        ],
        out_specs=pl.BlockSpec((block_rows, D), lambda i: (i, 0)),
    )(p0, p1, w, b)


def _tc_add(p0, p1, block_rows=512):
    M, D = p0.shape

    def body(a_ref, b_ref, o_ref):
        o_ref[...] = a_ref[...] + b_ref[...]

    return pl.pallas_call(
        body,
        out_shape=jax.ShapeDtypeStruct((M, D), jnp.float32),
        grid=(M // block_rows,),
        in_specs=[
            pl.BlockSpec((block_rows, D), lambda i: (i, 0)),
            pl.BlockSpec((block_rows, D), lambda i: (i, 0)),
        ],
        out_specs=pl.BlockSpec((block_rows, D), lambda i: (i, 0)),
    )(p0, p1)


def _sc_segment_sum(hw, edges3d, zeros_np, NP, n_chunks_per_tile):
    D = hw.shape[1]
    NCH = n_chunks_per_tile
    T = NCH // STEPS
    assert NCH == T * STEPS
    rows_per_tile = NP // N_TILES
    mesh = plsc.VectorSubcoreMesh(core_axis_name="c", subcore_axis_name="s")

    @functools.partial(
        pl.kernel,
        out_type=jax.ShapeDtypeStruct((N_CORES, NP, D), jnp.float32),
        mesh=mesh,
        scratch_types=[
            pltpu.VMEM((GRP, 2, CHUNK), jnp.int32),
            pltpu.VMEM((GRP, 2, CHUNK), jnp.int32),
            pltpu.VMEM((CHUNK, D), jnp.float32),
            pltpu.VMEM((CHUNK, D), jnp.float32),
            pltpu.VMEM_SHARED((NP, D), jnp.float32),
            pltpu.SemaphoreType.DMA,
            pltpu.SemaphoreType.DMA,
            pltpu.SemaphoreType.DMA,
            pltpu.SemaphoreType.DMA,
            pltpu.SemaphoreType.DMA,
            pltpu.SemaphoreType.DMA,
        ],
        compiler_params=pltpu.CompilerParams(use_tc_tiling_on_sc=False),
    )
    def k(hw_hbm, edges_hbm, zeros_hbm, out_hbm, idxA, idxB, rows0, rows1,
          acc, gsem0, gsem1, ssem0, ssem1, isemA, isemB):
        cid = lax.axis_index("c")
        sid = lax.axis_index("s")
        wid = sid * N_CORES + cid
        r0 = sid * rows_per_tile
        base = wid * NCH
        rows = (rows0, rows1)
        gsems = (gsem0, gsem1)
        ssems = (ssem0, ssem1)

        pltpu.sync_copy(
            zeros_hbm.at[pl.ds(r0, rows_per_tile)],
            acc.at[pl.ds(r0, rows_per_tile)],
        )
        plsc.subcore_barrier()

        def start_g(idx, row, p):
            pltpu.async_copy(hw_hbm.at[idx.at[row, 0]], rows[p], gsems[p])

        def wait_g(idx, row, p):
            pltpu.make_async_copy(
                hw_hbm.at[idx.at[row, 0]], rows[p], gsems[p]
            ).wait()

        def start_s(idx, row, p):
            pltpu.async_copy(
                rows[p], acc.at[idx.at[row, 1]], ssems[p], add=True
            )

        def wait_s(idx, row, p):
            pltpu.make_async_copy(
                rows[p], acc.at[idx.at[row, 1]], ssems[p]
            ).wait()

        pltpu.sync_copy(edges_hbm.at[pl.ds(base, GRP)], idxA)
        start_g(idxA, 0, 0)

        def outer(t, carry):
            for u in range(STEPS):
                p = u % 2
                q = 1 - p
                idx_cur = idxA if u < GRP else idxB
                row_cur = u % GRP
                wait_g(idx_cur, row_cur, p)
                start_s(idx_cur, row_cur, p)
                if u == 0:
                    @pl.when(t > 0)
                    def _():
                        wait_s(idxB, GRP - 1, q)
                else:
                    pidx = idxA if (u - 1) < GRP else idxB
                    wait_s(pidx, (u - 1) % GRP, q)
                if u == 1:
                    pltpu.async_copy(
                        edges_hbm.at[pl.ds(base + t * STEPS + GRP, GRP)],
                        idxB, isemB,
                    )
                if u < GRP - 1:
                    start_g(idxA, u + 1, q)
                elif u == GRP - 1:
                    pltpu.make_async_copy(
                        edges_hbm.at[pl.ds(base + t * STEPS + GRP, GRP)],
                        idxB, isemB,
                    ).wait()
                    start_g(idxB, 0, q)
                elif u < STEPS - 1:
                    start_g(idxB, u - GRP + 1, q)
                else:
                    @pl.when(t + 1 < T)
                    def _():
                        pltpu.make_async_copy(
                            edges_hbm.at[pl.ds(base + (t + 1) * STEPS, GRP)],
                            idxA, isemA,
                        ).wait()
                        start_g(idxA, 0, q)
                if u == GRP + 1:
                    @pl.when(t + 1 < T)
                    def _():
                        pltpu.async_copy(
                            edges_hbm.at[pl.ds(base + (t + 1) * STEPS, GRP)],
                            idxA, isemA,
                        )
            return carry

        lax.fori_loop(0, T, outer, 0)
        wait_s(idxB, GRP - 1, 1)
        plsc.subcore_barrier()
        pltpu.sync_copy(
            acc.at[pl.ds(r0, rows_per_tile)],
            out_hbm.at[cid, pl.ds(r0, rows_per_tile)],
        )

    return k(hw, edges3d, zeros_np)


def _sc_decoder(h2, pairs3d, n_chunks_per_tile):
    NP, D = h2.shape
    CH = pairs3d.shape[0]
    mesh = plsc.VectorSubcoreMesh(core_axis_name="c", subcore_axis_name="s")

    NCH = n_chunks_per_tile
    assert NCH % 2 == 0

    @functools.partial(
        pl.kernel,
        out_type=jax.ShapeDtypeStruct((CH * CHUNK,), jnp.float32),
        mesh=mesh,
        scratch_types=[
            pltpu.VMEM((NCH, 2, CHUNK), jnp.int32),
            pltpu.VMEM((2, CHUNK, D), jnp.float32),
            pltpu.VMEM((2, CHUNK, D), jnp.float32),
            pltpu.VMEM((CHUNK,), jnp.float32),
            pltpu.SemaphoreType.DMA,
            pltpu.SemaphoreType.DMA,
        ],
        compiler_params=pltpu.CompilerParams(
            use_tc_tiling_on_sc=False, needs_layout_passes=False
        ),
    )
    def k(h_hbm, pairs_hbm, out_hbm, idx_all, ra, rb, res_v, sem0, sem1):
        cid = lax.axis_index("c")
        sid = lax.axis_index("s")
        wid = sid * N_CORES + cid
        base = wid * NCH

        lane = lax.iota(jnp.int32, 16)
        pltpu.sync_copy(pairs_hbm.at[pl.ds(base, NCH)], idx_all)

        def start_gather(j, buf, sem):
            pltpu.async_copy(h_hbm.at[idx_all.at[j, 0]], ra.at[buf], sem)
            pltpu.async_copy(h_hbm.at[idx_all.at[j, 1]], rb.at[buf], sem)

        def drain_gather(j, buf, sem):
            pltpu.make_async_copy(
                h_hbm.at[idx_all.at[j, 0]], ra.at[buf], sem
            ).wait()
            pltpu.make_async_copy(
                h_hbm.at[idx_all.at[j, 1]], rb.at[buf], sem
            ).wait()

        def compute(j, buf):
            def group_body(g, c2):
                res = jnp.zeros((16,), jnp.float32)
                for p2 in range(16):
                    p = g * 16 + p2
                    acc = ra[buf, p, pl.ds(0, 16)] * rb[buf, p, pl.ds(0, 16)]
                    acc = acc + ra[buf, p, pl.ds(16, 16)] * rb[buf, p, pl.ds(16, 16)]
                    acc = acc + ra[buf, p, pl.ds(32, 16)] * rb[buf, p, pl.ds(32, 16)]
                    acc = acc + ra[buf, p, pl.ds(48, 16)] * rb[buf, p, pl.ds(48, 16)]
                    res = jnp.where(lane == p2, jnp.sum(acc), res)
                res_v[pl.ds(g * 16, 16)] = res
                return c2

            lax.fori_loop(0, CHUNK // 16, group_body, 0)
            pltpu.sync_copy(
                res_v, out_hbm.at[pl.ds((base + j) * CHUNK, CHUNK)]
            )

        start_gather(0, 0, sem0)

        def body(k2, carry):
            j = 2 * k2
            start_gather(j + 1, 1, sem1)
            drain_gather(j, 0, sem0)
            compute(j, 0)

            @pl.when(k2 + 1 < NCH // 2)
            def _():
                start_gather(j + 2, 0, sem0)

            drain_gather(j + 1, 1, sem1)
            compute(j + 1, 1)
            return carry

        lax.fori_loop(0, NCH // 2, body, 0)

    return k(h2, pairs3d)


def _pad_edges(src, dst, n_real_rows, n_pad_rows):
    E = src.shape[0]
    round_sz = STEPS * NW * CHUNK
    Epad = ((E + round_sz - 1) // round_sz) * round_sz
    pad = Epad - E
    if pad:
        ar = jnp.arange(pad, dtype=jnp.int32)
        src = jnp.concatenate([src, (ar * 97) % n_real_rows])
        dst = jnp.concatenate([dst, n_real_rows + ar % n_pad_rows])
    edges3d = jnp.stack(
        [src.reshape(-1, CHUNK), dst.reshape(-1, CHUNK)], axis=1
    )
    return edges3d, Epad


def kernel(features, edge_index, positive_edge_pairs, negative_edge_pairs,
           W1, b1, W2, b2):
    N, D_IN = features.shape
    H1 = W1.shape[1]
    H2 = W2.shape[1]

    NP = 10240
    n_pad_rows = NP - N

    src = edge_index[0].astype(jnp.int32)
    dst = edge_index[1].astype(jnp.int32)
    edges3d, Epad = _pad_edges(src, dst, N, n_pad_rows)
    n_chunks_per_tile = Epad // (NW * CHUNK)

    z1 = jnp.zeros((NP, H1), jnp.float32)
    z2 = jnp.zeros((NP, H2), jnp.float32)

    hw1 = _tc_matmul_bias(features, W1, b1.reshape(1, H1), block_rows=400)
    parts1 = _sc_segment_sum(hw1, edges3d, z1, NP, n_chunks_per_tile)

    hw2 = _tc_relu_add_matmul_bias(parts1[0], parts1[1], W2, b2.reshape(1, H2))
    parts2 = _sc_segment_sum(hw2, edges3d, z2, NP, n_chunks_per_tile)
    h2 = _tc_add(parts2[0], parts2[1])

    pa = jnp.concatenate(
        [positive_edge_pairs[0], negative_edge_pairs[0]]
    ).astype(jnp.int32)
    pb = jnp.concatenate(
        [positive_edge_pairs[1], negative_edge_pairs[1]]
    ).astype(jnp.int32)
    P = pa.shape[0]
    round_sz = NW * CHUNK
    Ppad = ((P + round_sz - 1) // round_sz) * round_sz
    padp = Ppad - P
    if padp:
        ar = jnp.arange(padp, dtype=jnp.int32)
        pa = jnp.concatenate([pa, (ar * 131) % N])
        pb = jnp.concatenate([pb, (ar * 173) % N])
    pairs3d = jnp.stack([pa.reshape(-1, CHUNK), pb.reshape(-1, CHUNK)], axis=1)

    result = _sc_decoder(h2, pairs3d, Ppad // (NW * CHUNK))
    return result[:P]


# speedup vs baseline: 1.1365x; 1.1365x over previous
import functools

import jax
import jax.numpy as jnp
from jax import lax
from jax.experimental import pallas as pl
from jax.experimental.pallas import tpu as pltpu
from jax.experimental.pallas import tpu_sc as plsc

N_TILES = 16
N_CORES = 2
NW = N_TILES * N_CORES
CHUNK = 128
GRP = 8
STEPS = 2 * GRP


def _tc_matmul_bias(x, w, b, block_rows):
    M, K = x.shape
    D = w.shape[1]

    def body(x_ref, w_ref, b_ref, o_ref):
        o_ref[...] = (
            jnp.dot(x_ref[...], w_ref[...], preferred_element_type=jnp.float32)
            + b_ref[...]
        )

    return pl.pallas_call(
        body,
        out_shape=jax.ShapeDtypeStruct((M, D), jnp.float32),
        grid=(M // block_rows,),
        in_specs=[
            pl.BlockSpec((block_rows, K), lambda i: (i, 0)),
            pl.BlockSpec((K, D), lambda i: (0, 0)),
            pl.BlockSpec((1, D), lambda i: (0, 0)),
        ],
        out_specs=pl.BlockSpec((block_rows, D), lambda i: (i, 0)),
    )(x, w, b)


def _tc_relu_add_matmul_bias(p0, p1, w, b, block_rows=512):
    M, K = p0.shape
    D = w.shape[1]

    def body(p0_ref, p1_ref, w_ref, b_ref, o_ref):
        h = jnp.maximum(p0_ref[...] + p1_ref[...], 0.0)
        o_ref[...] = (
            jnp.dot(h, w_ref[...], preferred_element_type=jnp.float32) + b_ref[...]
        )

    return pl.pallas_call(
        body,
        out_shape=jax.ShapeDtypeStruct((M, D), jnp.float32),
        grid=(M // block_rows,),
        in_specs=[
            pl.BlockSpec((block_rows, K), lambda i: (i, 0)),
            pl.BlockSpec((block_rows, K), lambda i: (i, 0)),
            pl.BlockSpec((K, D), lambda i: (0, 0)),
            pl.BlockSpec((1, D), lambda i: (0, 0)),
        ],
        out_specs=pl.BlockSpec((block_rows, D), lambda i: (i, 0)),
    )(p0, p1, w, b)


def _tc_add(p0, p1, block_rows=512):
    M, D = p0.shape

    def body(a_ref, b_ref, o_ref):
        o_ref[...] = a_ref[...] + b_ref[...]

    return pl.pallas_call(
        body,
        out_shape=jax.ShapeDtypeStruct((M, D), jnp.float32),
        grid=(M // block_rows,),
        in_specs=[
            pl.BlockSpec((block_rows, D), lambda i: (i, 0)),
            pl.BlockSpec((block_rows, D), lambda i: (i, 0)),
        ],
        out_specs=pl.BlockSpec((block_rows, D), lambda i: (i, 0)),
    )(p0, p1)


def _sc_segment_sum(hw, edges3d, zeros_np, NP, n_chunks_per_tile):
    D = hw.shape[1]
    NCH = n_chunks_per_tile
    T = NCH // STEPS
    assert NCH == T * STEPS
    rows_per_tile = NP // N_TILES
    mesh = plsc.VectorSubcoreMesh(core_axis_name="c", subcore_axis_name="s")

    @functools.partial(
        pl.kernel,
        out_type=jax.ShapeDtypeStruct((N_CORES, NP, D), jnp.float32),
        mesh=mesh,
        scratch_types=[
            pltpu.VMEM((GRP, 2, CHUNK), jnp.int32),
            pltpu.VMEM((GRP, 2, CHUNK), jnp.int32),
            pltpu.VMEM((CHUNK, D), jnp.float32),
            pltpu.VMEM((CHUNK, D), jnp.float32),
            pltpu.VMEM_SHARED((NP, D), jnp.float32),
            pltpu.SemaphoreType.DMA,
            pltpu.SemaphoreType.DMA,
            pltpu.SemaphoreType.DMA,
            pltpu.SemaphoreType.DMA,
        ],
        compiler_params=pltpu.CompilerParams(use_tc_tiling_on_sc=False),
    )
    def k(hw_hbm, edges_hbm, zeros_hbm, out_hbm, idxA, idxB, rows0, rows1,
          acc, gsem0, gsem1, isemA, isemB):
        cid = lax.axis_index("c")
        sid = lax.axis_index("s")
        wid = sid * N_CORES + cid
        r0 = sid * rows_per_tile
        base = wid * NCH
        rows = (rows0, rows1)
        gsems = (gsem0, gsem1)

        pltpu.sync_copy(
            zeros_hbm.at[pl.ds(r0, rows_per_tile)],
            acc.at[pl.ds(r0, rows_per_tile)],
        )
        plsc.subcore_barrier()

        def start_g(idx, row, p):
            pltpu.async_copy(hw_hbm.at[idx.at[row, 0]], rows[p], gsems[p])

        def wait_g(idx, row, p):
            pltpu.make_async_copy(
                hw_hbm.at[idx.at[row, 0]], rows[p], gsems[p]
            ).wait()

        def scat(idx, row, p):
            pltpu.sync_copy(rows[p], acc.at[idx.at[row, 1]], add=True)

        pltpu.sync_copy(edges_hbm.at[pl.ds(base, GRP)], idxA)
        start_g(idxA, 0, 0)

        def outer(t, carry):
            for u in range(STEPS):
                p = u % 2
                q = 1 - p
                idx_cur = idxA if u < GRP else idxB
                row_cur = u % GRP
                if u == 0:
                    pltpu.async_copy(
                        edges_hbm.at[pl.ds(base + t * STEPS + GRP, GRP)],
                        idxB, isemB,
                    )
                if u == GRP:
                    @pl.when(t + 1 < T)
                    def _():
                        pltpu.async_copy(
                            edges_hbm.at[pl.ds(base + (t + 1) * STEPS, GRP)],
                            idxA, isemA,
                        )
                if u < GRP - 1:
                    start_g(idxA, u + 1, q)
                elif u == GRP - 1:
                    pltpu.make_async_copy(
                        edges_hbm.at[pl.ds(base + t * STEPS + GRP, GRP)],
                        idxB, isemB,
                    ).wait()
                    start_g(idxB, 0, q)
                elif u < STEPS - 1:
                    start_g(idxB, u - GRP + 1, q)
                else:
                    @pl.when(t + 1 < T)
                    def _():
                        pltpu.make_async_copy(
                            edges_hbm.at[pl.ds(base + (t + 1) * STEPS, GRP)],
                            idxA, isemA,
                        ).wait()
                        start_g(idxA, 0, q)
                wait_g(idx_cur, row_cur, p)
                scat(idx_cur, row_cur, p)
            return carry

        lax.fori_loop(0, T, outer, 0)
        plsc.subcore_barrier()
        pltpu.sync_copy(
            acc.at[pl.ds(r0, rows_per_tile)],
            out_hbm.at[cid, pl.ds(r0, rows_per_tile)],
        )

    return k(hw, edges3d, zeros_np)


def _sc_decoder(h2, pairs3d, n_chunks_per_tile):
    NP, D = h2.shape
    CH = pairs3d.shape[0]
    mesh = plsc.VectorSubcoreMesh(core_axis_name="c", subcore_axis_name="s")

    NCH = n_chunks_per_tile
    assert NCH % 2 == 0

    @functools.partial(
        pl.kernel,
        out_type=jax.ShapeDtypeStruct((CH * CHUNK,), jnp.float32),
        mesh=mesh,
        scratch_types=[
            pltpu.VMEM((NCH, 2, CHUNK), jnp.int32),
            pltpu.VMEM((2, CHUNK, D), jnp.float32),
            pltpu.VMEM((2, CHUNK, D), jnp.float32),
            pltpu.VMEM((CHUNK,), jnp.float32),
            pltpu.SemaphoreType.DMA,
            pltpu.SemaphoreType.DMA,
        ],
        compiler_params=pltpu.CompilerParams(
            use_tc_tiling_on_sc=False, needs_layout_passes=False
        ),
    )
    def k(h_hbm, pairs_hbm, out_hbm, idx_all, ra, rb, res_v, sem0, sem1):
        cid = lax.axis_index("c")
        sid = lax.axis_index("s")
        wid = sid * N_CORES + cid
        base = wid * NCH

        lane = lax.iota(jnp.int32, 16)
        pltpu.sync_copy(pairs_hbm.at[pl.ds(base, NCH)], idx_all)

        def start_gather(j, buf, sem):
            pltpu.async_copy(h_hbm.at[idx_all.at[j, 0]], ra.at[buf], sem)
            pltpu.async_copy(h_hbm.at[idx_all.at[j, 1]], rb.at[buf], sem)

        def drain_gather(j, buf, sem):
            pltpu.make_async_copy(
                h_hbm.at[idx_all.at[j, 0]], ra.at[buf], sem
            ).wait()
            pltpu.make_async_copy(
                h_hbm.at[idx_all.at[j, 1]], rb.at[buf], sem
            ).wait()

        def compute(j, buf):
            def group_body(g, c2):
                res = jnp.zeros((16,), jnp.float32)
                for p2 in range(16):
                    p = g * 16 + p2
                    acc = ra[buf, p, pl.ds(0, 16)] * rb[buf, p, pl.ds(0, 16)]
                    acc = acc + ra[buf, p, pl.ds(16, 16)] * rb[buf, p, pl.ds(16, 16)]
                    acc = acc + ra[buf, p, pl.ds(32, 16)] * rb[buf, p, pl.ds(32, 16)]
                    acc = acc + ra[buf, p, pl.ds(48, 16)] * rb[buf, p, pl.ds(48, 16)]
                    res = jnp.where(lane == p2, jnp.sum(acc), res)
                res_v[pl.ds(g * 16, 16)] = res
                return c2

            lax.fori_loop(0, CHUNK // 16, group_body, 0)
            pltpu.sync_copy(
                res_v, out_hbm.at[pl.ds((base + j) * CHUNK, CHUNK)]
            )

        start_gather(0, 0, sem0)

        def body(k2, carry):
            j = 2 * k2
            start_gather(j + 1, 1, sem1)
            drain_gather(j, 0, sem0)
            compute(j, 0)

            @pl.when(k2 + 1 < NCH // 2)
            def _():
                start_gather(j + 2, 0, sem0)

            drain_gather(j + 1, 1, sem1)
            compute(j + 1, 1)
            return carry

        lax.fori_loop(0, NCH // 2, body, 0)

    return k(h2, pairs3d)


def _pad_edges(src, dst, n_real_rows, n_pad_rows):
    E = src.shape[0]
    round_sz = STEPS * NW * CHUNK
    Epad = ((E + round_sz - 1) // round_sz) * round_sz
    pad = Epad - E
    if pad:
        ar = jnp.arange(pad, dtype=jnp.int32)
        src = jnp.concatenate([src, (ar * 97) % n_real_rows])
        dst = jnp.concatenate([dst, n_real_rows + ar % n_pad_rows])
    edges3d = jnp.stack(
        [src.reshape(-1, CHUNK), dst.reshape(-1, CHUNK)], axis=1
    )
    return edges3d, Epad


def kernel(features, edge_index, positive_edge_pairs, negative_edge_pairs,
           W1, b1, W2, b2):
    N, D_IN = features.shape
    H1 = W1.shape[1]
    H2 = W2.shape[1]

    NP = 10240
    n_pad_rows = NP - N

    src = edge_index[0].astype(jnp.int32)
    dst = edge_index[1].astype(jnp.int32)
    edges3d, Epad = _pad_edges(src, dst, N, n_pad_rows)
    n_chunks_per_tile = Epad // (NW * CHUNK)

    z1 = jnp.zeros((NP, H1), jnp.float32)
    z2 = jnp.zeros((NP, H2), jnp.float32)

    hw1 = _tc_matmul_bias(features, W1, b1.reshape(1, H1), block_rows=400)
    parts1 = _sc_segment_sum(hw1, edges3d, z1, NP, n_chunks_per_tile)

    hw2 = _tc_relu_add_matmul_bias(parts1[0], parts1[1], W2, b2.reshape(1, H2))
    parts2 = _sc_segment_sum(hw2, edges3d, z2, NP, n_chunks_per_tile)
    h2 = _tc_add(parts2[0], parts2[1])

    pa = jnp.concatenate(
        [positive_edge_pairs[0], negative_edge_pairs[0]]
    ).astype(jnp.int32)
    pb = jnp.concatenate(
        [positive_edge_pairs[1], negative_edge_pairs[1]]
    ).astype(jnp.int32)
    P = pa.shape[0]
    round_sz = NW * CHUNK
    Ppad = ((P + round_sz - 1) // round_sz) * round_sz
    padp = Ppad - P
    if padp:
        ar = jnp.arange(padp, dtype=jnp.int32)
        pa = jnp.concatenate([pa, (ar * 131) % N])
        pb = jnp.concatenate([pb, (ar * 173) % N])
    pairs3d = jnp.stack([pa.reshape(-1, CHUNK), pb.reshape(-1, CHUNK)], axis=1)

    result = _sc_decoder(h2, pairs3d, Ppad // (NW * CHUNK))
    return result[:P]


# speedup vs baseline: 1.1536x; 1.0151x over previous
import functools

import jax
import jax.numpy as jnp
from jax import lax
from jax.experimental import pallas as pl
from jax.experimental.pallas import tpu as pltpu
from jax.experimental.pallas import tpu_sc as plsc

N_TILES = 16
N_CORES = 2
NW = N_TILES * N_CORES
CHUNK = 128
GRP = 8
STEPS = 2 * GRP


def _tc_matmul_bias(x, w, b, block_rows):
    M, K = x.shape
    D = w.shape[1]

    def body(x_ref, w_ref, b_ref, o_ref):
        o_ref[...] = (
            jnp.dot(x_ref[...], w_ref[...], preferred_element_type=jnp.float32)
            + b_ref[...]
        )

    return pl.pallas_call(
        body,
        out_shape=jax.ShapeDtypeStruct((M, D), jnp.float32),
        grid=(M // block_rows,),
        in_specs=[
            pl.BlockSpec((block_rows, K), lambda i: (i, 0)),
            pl.BlockSpec((K, D), lambda i: (0, 0)),
            pl.BlockSpec((1, D), lambda i: (0, 0)),
        ],
        out_specs=pl.BlockSpec((block_rows, D), lambda i: (i, 0)),
    )(x, w, b)


def _tc_relu_add_matmul_bias(p0, p1, w, b, block_rows=512):
    M, K = p0.shape
    D = w.shape[1]

    def body(p0_ref, p1_ref, w_ref, b_ref, o_ref):
        h = jnp.maximum(p0_ref[...] + p1_ref[...], 0.0)
        o_ref[...] = (
            jnp.dot(h, w_ref[...], preferred_element_type=jnp.float32) + b_ref[...]
        )

    return pl.pallas_call(
        body,
        out_shape=jax.ShapeDtypeStruct((M, D), jnp.float32),
        grid=(M // block_rows,),
        in_specs=[
            pl.BlockSpec((block_rows, K), lambda i: (i, 0)),
            pl.BlockSpec((block_rows, K), lambda i: (i, 0)),
            pl.BlockSpec((K, D), lambda i: (0, 0)),
            pl.BlockSpec((1, D), lambda i: (0, 0)),
        ],
        out_specs=pl.BlockSpec((block_rows, D), lambda i: (i, 0)),
    )(p0, p1, w, b)


def _tc_add(p0, p1, block_rows=512):
    M, D = p0.shape

    def body(a_ref, b_ref, o_ref):
        o_ref[...] = a_ref[...] + b_ref[...]

    return pl.pallas_call(
        body,
        out_shape=jax.ShapeDtypeStruct((M, D), jnp.float32),
        grid=(M // block_rows,),
        in_specs=[
            pl.BlockSpec((block_rows, D), lambda i: (i, 0)),
            pl.BlockSpec((block_rows, D), lambda i: (i, 0)),
        ],
        out_specs=pl.BlockSpec((block_rows, D), lambda i: (i, 0)),
    )(p0, p1)


def _sc_segment_sum(hw, edges3d, zeros_np, NP, n_chunks_per_tile,
                    tc_tiling=True):
    D = hw.shape[1]
    NCH = n_chunks_per_tile
    T = NCH // STEPS
    assert NCH == T * STEPS
    rows_per_tile = NP // N_TILES
    mesh = plsc.VectorSubcoreMesh(core_axis_name="c", subcore_axis_name="s")

    @functools.partial(
        pl.kernel,
        out_type=[
            jax.ShapeDtypeStruct((NP, D), jnp.float32),
            jax.ShapeDtypeStruct((NP, D), jnp.float32),
        ],
        mesh=mesh,
        scratch_types=[
            pltpu.VMEM((GRP, 2, CHUNK), jnp.int32),
            pltpu.VMEM((GRP, 2, CHUNK), jnp.int32),
            pltpu.VMEM((CHUNK, D), jnp.float32),
            pltpu.VMEM((CHUNK, D), jnp.float32),
            pltpu.VMEM_SHARED((NP, D), jnp.float32),
            pltpu.SemaphoreType.DMA,
            pltpu.SemaphoreType.DMA,
            pltpu.SemaphoreType.DMA,
            pltpu.SemaphoreType.DMA,
        ],
        compiler_params=pltpu.CompilerParams(use_tc_tiling_on_sc=tc_tiling),
    )
    def k(hw_hbm, edges_hbm, zeros_hbm, out0_hbm, out1_hbm, idxA, idxB,
          rows0, rows1, acc, gsem0, gsem1, isemA, isemB):
        cid = lax.axis_index("c")
        sid = lax.axis_index("s")
        wid = sid * N_CORES + cid
        r0 = sid * rows_per_tile
        base = wid * NCH
        rows = (rows0, rows1)
        gsems = (gsem0, gsem1)

        pltpu.sync_copy(
            zeros_hbm.at[pl.ds(r0, rows_per_tile)],
            acc.at[pl.ds(r0, rows_per_tile)],
        )
        plsc.subcore_barrier()

        def start_g(idx, row, p):
            pltpu.async_copy(hw_hbm.at[idx.at[row, 0]], rows[p], gsems[p])

        def wait_g(idx, row, p):
            pltpu.make_async_copy(
                hw_hbm.at[idx.at[row, 0]], rows[p], gsems[p]
            ).wait()

        def scat(idx, row, p):
            pltpu.sync_copy(rows[p], acc.at[idx.at[row, 1]], add=True)

        pltpu.sync_copy(edges_hbm.at[pl.ds(base, GRP)], idxA)
        start_g(idxA, 0, 0)

        def outer(t, carry):
            for u in range(STEPS):
                p = u % 2
                q = 1 - p
                idx_cur = idxA if u < GRP else idxB
                row_cur = u % GRP
                if u == 0:
                    pltpu.async_copy(
                        edges_hbm.at[pl.ds(base + t * STEPS + GRP, GRP)],
                        idxB, isemB,
                    )
                if u == GRP:
                    @pl.when(t + 1 < T)
                    def _():
                        pltpu.async_copy(
                            edges_hbm.at[pl.ds(base + (t + 1) * STEPS, GRP)],
                            idxA, isemA,
                        )
                if u < GRP - 1:
                    start_g(idxA, u + 1, q)
                elif u == GRP - 1:
                    pltpu.make_async_copy(
                        edges_hbm.at[pl.ds(base + t * STEPS + GRP, GRP)],
                        idxB, isemB,
                    ).wait()
                    start_g(idxB, 0, q)
                elif u < STEPS - 1:
                    start_g(idxB, u - GRP + 1, q)
                else:
                    @pl.when(t + 1 < T)
                    def _():
                        pltpu.make_async_copy(
                            edges_hbm.at[pl.ds(base + (t + 1) * STEPS, GRP)],
                            idxA, isemA,
                        ).wait()
                        start_g(idxA, 0, q)
                wait_g(idx_cur, row_cur, p)
                scat(idx_cur, row_cur, p)
            return carry

        lax.fori_loop(0, T, outer, 0)
        plsc.subcore_barrier()

        @pl.when(cid == 0)
        def _():
            pltpu.sync_copy(
                acc.at[pl.ds(r0, rows_per_tile)],
                out0_hbm.at[pl.ds(r0, rows_per_tile)],
            )

        @pl.when(cid == 1)
        def _():
            pltpu.sync_copy(
                acc.at[pl.ds(r0, rows_per_tile)],
                out1_hbm.at[pl.ds(r0, rows_per_tile)],
            )

    return k(hw, edges3d, zeros_np)


def _sc_decoder(q0, q1, pairs3d, n_chunks_per_tile):
    NP, D = q0.shape
    CH = pairs3d.shape[0]
    mesh = plsc.VectorSubcoreMesh(core_axis_name="c", subcore_axis_name="s")

    NCH = n_chunks_per_tile
    assert NCH % 2 == 0

    @functools.partial(
        pl.kernel,
        out_type=jax.ShapeDtypeStruct((CH * CHUNK,), jnp.float32),
        mesh=mesh,
        scratch_types=[
            pltpu.VMEM((NCH, 2, CHUNK), jnp.int32),
            pltpu.VMEM((2, CHUNK, D), jnp.float32),
            pltpu.VMEM((2, CHUNK, D), jnp.float32),
            pltpu.VMEM((2, CHUNK, D), jnp.float32),
            pltpu.VMEM((2, CHUNK, D), jnp.float32),
            pltpu.VMEM((CHUNK,), jnp.float32),
            pltpu.SemaphoreType.DMA,
            pltpu.SemaphoreType.DMA,
        ],
        compiler_params=pltpu.CompilerParams(
            use_tc_tiling_on_sc=False, needs_layout_passes=False
        ),
    )
    def k(q0_hbm, q1_hbm, pairs_hbm, out_hbm, idx_all, ra0, ra1, rb0, rb1,
          res_v, sem0, sem1):
        cid = lax.axis_index("c")
        sid = lax.axis_index("s")
        wid = sid * N_CORES + cid
        base = wid * NCH

        lane = lax.iota(jnp.int32, 16)
        pltpu.sync_copy(pairs_hbm.at[pl.ds(base, NCH)], idx_all)

        def start_gather(j, buf, sem):
            pltpu.async_copy(q0_hbm.at[idx_all.at[j, 0]], ra0.at[buf], sem)
            pltpu.async_copy(q1_hbm.at[idx_all.at[j, 0]], ra1.at[buf], sem)
            pltpu.async_copy(q0_hbm.at[idx_all.at[j, 1]], rb0.at[buf], sem)
            pltpu.async_copy(q1_hbm.at[idx_all.at[j, 1]], rb1.at[buf], sem)

        def drain_gather(j, buf, sem):
            pltpu.make_async_copy(
                q0_hbm.at[idx_all.at[j, 0]], ra0.at[buf], sem
            ).wait()
            pltpu.make_async_copy(
                q1_hbm.at[idx_all.at[j, 0]], ra1.at[buf], sem
            ).wait()
            pltpu.make_async_copy(
                q0_hbm.at[idx_all.at[j, 1]], rb0.at[buf], sem
            ).wait()
            pltpu.make_async_copy(
                q1_hbm.at[idx_all.at[j, 1]], rb1.at[buf], sem
            ).wait()

        def compute(j, buf):
            def group_body(g, c2):
                res = jnp.zeros((16,), jnp.float32)
                for p2 in range(16):
                    p = g * 16 + p2
                    acc = None
                    for o in (0, 16, 32, 48):
                        a = (ra0[buf, p, pl.ds(o, 16)]
                             + ra1[buf, p, pl.ds(o, 16)])
                        b = (rb0[buf, p, pl.ds(o, 16)]
                             + rb1[buf, p, pl.ds(o, 16)])
                        acc = a * b if acc is None else acc + a * b
                    res = jnp.where(lane == p2, jnp.sum(acc), res)
                res_v[pl.ds(g * 16, 16)] = res
                return c2

            lax.fori_loop(0, CHUNK // 16, group_body, 0)
            pltpu.sync_copy(
                res_v, out_hbm.at[pl.ds((base + j) * CHUNK, CHUNK)]
            )

        start_gather(0, 0, sem0)

        def body(k2, carry):
            j = 2 * k2
            start_gather(j + 1, 1, sem1)
            drain_gather(j, 0, sem0)
            compute(j, 0)

            @pl.when(k2 + 1 < NCH // 2)
            def _():
                start_gather(j + 2, 0, sem0)

            drain_gather(j + 1, 1, sem1)
            compute(j + 1, 1)
            return carry

        lax.fori_loop(0, NCH // 2, body, 0)

    return k(q0, q1, pairs3d)


def _pad_edges(src, dst, n_real_rows, n_pad_rows):
    E = src.shape[0]
    round_sz = STEPS * NW * CHUNK
    Epad = ((E + round_sz - 1) // round_sz) * round_sz
    pad = Epad - E
    if pad:
        ar = jnp.arange(pad, dtype=jnp.int32)
        src = jnp.concatenate([src, (ar * 97) % n_real_rows])
        dst = jnp.concatenate([dst, n_real_rows + ar % n_pad_rows])
    edges3d = jnp.stack(
        [src.reshape(-1, CHUNK), dst.reshape(-1, CHUNK)], axis=1
    )
    return edges3d, Epad


def kernel(features, edge_index, positive_edge_pairs, negative_edge_pairs,
           W1, b1, W2, b2):
    N, D_IN = features.shape
    H1 = W1.shape[1]
    H2 = W2.shape[1]

    NP = 10240
    n_pad_rows = NP - N

    src = edge_index[0].astype(jnp.int32)
    dst = edge_index[1].astype(jnp.int32)
    edges3d, Epad = _pad_edges(src, dst, N, n_pad_rows)
    n_chunks_per_tile = Epad // (NW * CHUNK)

    z1 = jnp.zeros((NP, H1), jnp.float32)
    z2 = jnp.zeros((NP, H2), jnp.float32)

    hw1 = _tc_matmul_bias(features, W1, b1.reshape(1, H1), block_rows=400)
    p10, p11 = _sc_segment_sum(hw1, edges3d, z1, NP, n_chunks_per_tile,
                               tc_tiling=True)

    hw2 = _tc_relu_add_matmul_bias(p10, p11, W2, b2.reshape(1, H2))
    p20, p21 = _sc_segment_sum(hw2, edges3d, z2, NP, n_chunks_per_tile,
                               tc_tiling=False)

    pa = jnp.concatenate(
        [positive_edge_pairs[0], negative_edge_pairs[0]]
    ).astype(jnp.int32)
    pb = jnp.concatenate(
        [positive_edge_pairs[1], negative_edge_pairs[1]]
    ).astype(jnp.int32)
    P = pa.shape[0]
    round_sz = NW * CHUNK
    Ppad = ((P + round_sz - 1) // round_sz) * round_sz
    padp = Ppad - P
    if padp:
        ar = jnp.arange(padp, dtype=jnp.int32)
        pa = jnp.concatenate([pa, (ar * 131) % N])
        pb = jnp.concatenate([pb, (ar * 173) % N])
    pairs3d = jnp.stack([pa.reshape(-1, CHUNK), pb.reshape(-1, CHUNK)], axis=1)

    result = _sc_decoder(p20, p21, pairs3d, Ppad // (NW * CHUNK))
    return result[:P]


# speedup vs baseline: 1.2137x; 1.0521x over previous
import functools

import jax
import jax.numpy as jnp
from jax import lax
from jax.experimental import pallas as pl
from jax.experimental.pallas import tpu as pltpu
from jax.experimental.pallas import tpu_sc as plsc

N_TILES = 16
N_CORES = 2
NW = N_TILES * N_CORES
CHUNK = 128
GRP = 8
STEPS = 2 * GRP


def _tc_matmul_bias(x, w, b, block_rows):
    M, K = x.shape
    D = w.shape[1]

    def body(x_ref, w_ref, b_ref, o_ref):
        o_ref[...] = (
            jnp.dot(x_ref[...], w_ref[...], preferred_element_type=jnp.float32)
            + b_ref[...]
        )

    return pl.pallas_call(
        body,
        out_shape=jax.ShapeDtypeStruct((M, D), jnp.float32),
        grid=(M // block_rows,),
        in_specs=[
            pl.BlockSpec((block_rows, K), lambda i: (i, 0)),
            pl.BlockSpec((K, D), lambda i: (0, 0)),
            pl.BlockSpec((1, D), lambda i: (0, 0)),
        ],
        out_specs=pl.BlockSpec((block_rows, D), lambda i: (i, 0)),
    )(x, w, b)


def _tc_relu_add_matmul_bias(p0, p1, w, b, block_rows=512):
    M, K = p0.shape
    D = w.shape[1]

    def body(p0_ref, p1_ref, w_ref, b_ref, o_ref):
        h = jnp.maximum(p0_ref[...] + p1_ref[...], 0.0)
        o_ref[...] = (
            jnp.dot(h, w_ref[...], preferred_element_type=jnp.float32) + b_ref[...]
        )

    return pl.pallas_call(
        body,
        out_shape=jax.ShapeDtypeStruct((M, D), jnp.float32),
        grid=(M // block_rows,),
        in_specs=[
            pl.BlockSpec((block_rows, K), lambda i: (i, 0)),
            pl.BlockSpec((block_rows, K), lambda i: (i, 0)),
            pl.BlockSpec((K, D), lambda i: (0, 0)),
            pl.BlockSpec((1, D), lambda i: (0, 0)),
        ],
        out_specs=pl.BlockSpec((block_rows, D), lambda i: (i, 0)),
    )(p0, p1, w, b)


def _tc_add(p0, p1, block_rows=512):
    M, D = p0.shape

    def body(a_ref, b_ref, o_ref):
        o_ref[...] = a_ref[...] + b_ref[...]

    return pl.pallas_call(
        body,
        out_shape=jax.ShapeDtypeStruct((M, D), jnp.float32),
        grid=(M // block_rows,),
        in_specs=[
            pl.BlockSpec((block_rows, D), lambda i: (i, 0)),
            pl.BlockSpec((block_rows, D), lambda i: (i, 0)),
        ],
        out_specs=pl.BlockSpec((block_rows, D), lambda i: (i, 0)),
    )(p0, p1)


def _sc_segment_sum(hw, edges3d, zeros_np, NP, n_chunks_per_tile,
                    tc_tiling=True, stage_src=False):
    NH, D = hw.shape
    NCH = n_chunks_per_tile
    T = NCH // STEPS
    assert NCH == T * STEPS
    assert not stage_src or NH == NP
    rows_per_tile = NP // N_TILES
    mesh = plsc.VectorSubcoreMesh(core_axis_name="c", subcore_axis_name="s")

    scratch = [
        pltpu.VMEM((GRP, 2, CHUNK), jnp.int32),
        pltpu.VMEM((GRP, 2, CHUNK), jnp.int32),
        pltpu.VMEM((CHUNK, D), jnp.float32),
        pltpu.VMEM((CHUNK, D), jnp.float32),
        pltpu.VMEM_SHARED((NP, D), jnp.float32),
        pltpu.SemaphoreType.DMA,
        pltpu.SemaphoreType.DMA,
        pltpu.SemaphoreType.DMA,
        pltpu.SemaphoreType.DMA,
    ]
    if stage_src:
        scratch.append(pltpu.VMEM_SHARED((NP, D), jnp.float32))

    @functools.partial(
        pl.kernel,
        out_type=[
            jax.ShapeDtypeStruct((NP, D), jnp.float32),
            jax.ShapeDtypeStruct((NP, D), jnp.float32),
        ],
        mesh=mesh,
        scratch_types=scratch,
        compiler_params=pltpu.CompilerParams(use_tc_tiling_on_sc=tc_tiling),
    )
    def k(hw_hbm, edges_hbm, zeros_hbm, out0_hbm, out1_hbm, idxA, idxB,
          rows0, rows1, acc, gsem0, gsem1, isemA, isemB, *maybe_stage):
        cid = lax.axis_index("c")
        sid = lax.axis_index("s")
        wid = sid * N_CORES + cid
        r0 = sid * rows_per_tile
        base = wid * NCH
        rows = (rows0, rows1)
        gsems = (gsem0, gsem1)

        pltpu.sync_copy(
            zeros_hbm.at[pl.ds(r0, rows_per_tile)],
            acc.at[pl.ds(r0, rows_per_tile)],
        )
        if stage_src:
            src_ref = maybe_stage[0]
            pltpu.sync_copy(
                hw_hbm.at[pl.ds(r0, rows_per_tile)],
                src_ref.at[pl.ds(r0, rows_per_tile)],
            )
        else:
            src_ref = hw_hbm
        plsc.subcore_barrier()

        def start_g(idx, row, p):
            pltpu.async_copy(src_ref.at[idx.at[row, 0]], rows[p], gsems[p])

        def wait_g(idx, row, p):
            pltpu.make_async_copy(
                src_ref.at[idx.at[row, 0]], rows[p], gsems[p]
            ).wait()

        def scat(idx, row, p):
            pltpu.sync_copy(rows[p], acc.at[idx.at[row, 1]], add=True)

        pltpu.sync_copy(edges_hbm.at[pl.ds(base, GRP)], idxA)
        start_g(idxA, 0, 0)

        def outer(t, carry):
            for u in range(STEPS):
                p = u % 2
                q = 1 - p
                idx_cur = idxA if u < GRP else idxB
                row_cur = u % GRP
                if u == 0:
                    pltpu.async_copy(
                        edges_hbm.at[pl.ds(base + t * STEPS + GRP, GRP)],
                        idxB, isemB,
                    )
                if u == GRP:
                    @pl.when(t + 1 < T)
                    def _():
                        pltpu.async_copy(
                            edges_hbm.at[pl.ds(base + (t + 1) * STEPS, GRP)],
                            idxA, isemA,
                        )
                if u < GRP - 1:
                    start_g(idxA, u + 1, q)
                elif u == GRP - 1:
                    pltpu.make_async_copy(
                        edges_hbm.at[pl.ds(base + t * STEPS + GRP, GRP)],
                        idxB, isemB,
                    ).wait()
                    start_g(idxB, 0, q)
                elif u < STEPS - 1:
                    start_g(idxB, u - GRP + 1, q)
                else:
                    @pl.when(t + 1 < T)
                    def _():
                        pltpu.make_async_copy(
                            edges_hbm.at[pl.ds(base + (t + 1) * STEPS, GRP)],
                            idxA, isemA,
                        ).wait()
                        start_g(idxA, 0, q)
                wait_g(idx_cur, row_cur, p)
                scat(idx_cur, row_cur, p)
            return carry

        lax.fori_loop(0, T, outer, 0)
        plsc.subcore_barrier()

        @pl.when(cid == 0)
        def _():
            pltpu.sync_copy(
                acc.at[pl.ds(r0, rows_per_tile)],
                out0_hbm.at[pl.ds(r0, rows_per_tile)],
            )

        @pl.when(cid == 1)
        def _():
            pltpu.sync_copy(
                acc.at[pl.ds(r0, rows_per_tile)],
                out1_hbm.at[pl.ds(r0, rows_per_tile)],
            )

    return k(hw, edges3d, zeros_np)


def _sc_decoder(q0, q1, pairs3d, n_chunks_per_tile):
    NP, D = q0.shape
    CH = pairs3d.shape[0]
    mesh = plsc.VectorSubcoreMesh(core_axis_name="c", subcore_axis_name="s")

    NCH = n_chunks_per_tile
    assert NCH % 2 == 0
    rows_per_tile = NP // N_TILES
    n_stage_chunks = rows_per_tile // CHUNK

    @functools.partial(
        pl.kernel,
        out_type=jax.ShapeDtypeStruct((CH * CHUNK,), jnp.float32),
        mesh=mesh,
        scratch_types=[
            pltpu.VMEM((NCH, 2, CHUNK), jnp.int32),
            pltpu.VMEM((2, CHUNK, D), jnp.float32),
            pltpu.VMEM((2, CHUNK, D), jnp.float32),
            pltpu.VMEM((CHUNK,), jnp.float32),
            pltpu.VMEM((CHUNK, D), jnp.float32),
            pltpu.VMEM((n_stage_chunks, 1, CHUNK), jnp.int32),
            pltpu.VMEM_SHARED((NP, D), jnp.float32),
            pltpu.SemaphoreType.DMA,
            pltpu.SemaphoreType.DMA,
        ],
        compiler_params=pltpu.CompilerParams(
            use_tc_tiling_on_sc=False, needs_layout_passes=False
        ),
    )
    def k(q0_hbm, q1_hbm, pairs_hbm, out_hbm, idx_all, ra, rb,
          res_v, qbuf, iidx, h2s, sem0, sem1):
        cid = lax.axis_index("c")
        sid = lax.axis_index("s")
        wid = sid * N_CORES + cid
        base = wid * NCH
        r0 = sid * rows_per_tile

        lane = lax.iota(jnp.int32, 16)
        pltpu.sync_copy(pairs_hbm.at[pl.ds(base, NCH)], idx_all)

        for c in range(n_stage_chunks):
            for s in range(CHUNK // 16):
                iidx[c, 0, pl.ds(s * 16, 16)] = r0 + c * CHUNK + s * 16 + lane
        pltpu.sync_copy(
            q0_hbm.at[pl.ds(r0, rows_per_tile)],
            h2s.at[pl.ds(r0, rows_per_tile)],
        )
        for c in range(n_stage_chunks):
            pltpu.sync_copy(q1_hbm.at[pl.ds(r0 + c * CHUNK, CHUNK)], qbuf)
            pltpu.sync_copy(qbuf, h2s.at[iidx.at[c, 0]], add=True)
        plsc.subcore_barrier()

        def start_gather(j, buf, sem):
            pltpu.async_copy(h2s.at[idx_all.at[j, 0]], ra.at[buf], sem)
            pltpu.async_copy(h2s.at[idx_all.at[j, 1]], rb.at[buf], sem)

        def drain_gather(j, buf, sem):
            pltpu.make_async_copy(
                h2s.at[idx_all.at[j, 0]], ra.at[buf], sem
            ).wait()
            pltpu.make_async_copy(
                h2s.at[idx_all.at[j, 1]], rb.at[buf], sem
            ).wait()

        def compute(j, buf):
            def group_body(g, c2):
                res = jnp.zeros((16,), jnp.float32)
                for p2 in range(16):
                    p = g * 16 + p2
                    acc = ra[buf, p, pl.ds(0, 16)] * rb[buf, p, pl.ds(0, 16)]
                    acc = acc + ra[buf, p, pl.ds(16, 16)] * rb[buf, p, pl.ds(16, 16)]
                    acc = acc + ra[buf, p, pl.ds(32, 16)] * rb[buf, p, pl.ds(32, 16)]
                    acc = acc + ra[buf, p, pl.ds(48, 16)] * rb[buf, p, pl.ds(48, 16)]
                    res = jnp.where(lane == p2, jnp.sum(acc), res)
                res_v[pl.ds(g * 16, 16)] = res
                return c2

            lax.fori_loop(0, CHUNK // 16, group_body, 0)
            pltpu.sync_copy(
                res_v, out_hbm.at[pl.ds((base + j) * CHUNK, CHUNK)]
            )

        start_gather(0, 0, sem0)

        def body(k2, carry):
            j = 2 * k2
            start_gather(j + 1, 1, sem1)
            drain_gather(j, 0, sem0)
            compute(j, 0)

            @pl.when(k2 + 1 < NCH // 2)
            def _():
                start_gather(j + 2, 0, sem0)

            drain_gather(j + 1, 1, sem1)
            compute(j + 1, 1)
            return carry

        lax.fori_loop(0, NCH // 2, body, 0)

    return k(q0, q1, pairs3d)


def _pad_edges(src, dst, n_real_rows, n_pad_rows):
    E = src.shape[0]
    round_sz = STEPS * NW * CHUNK
    Epad = ((E + round_sz - 1) // round_sz) * round_sz
    pad = Epad - E
    if pad:
        ar = jnp.arange(pad, dtype=jnp.int32)
        src = jnp.concatenate([src, (ar * 97) % n_real_rows])
        dst = jnp.concatenate([dst, n_real_rows + ar % n_pad_rows])
    edges3d = jnp.stack(
        [src.reshape(-1, CHUNK), dst.reshape(-1, CHUNK)], axis=1
    )
    return edges3d, Epad


def kernel(features, edge_index, positive_edge_pairs, negative_edge_pairs,
           W1, b1, W2, b2):
    N, D_IN = features.shape
    H1 = W1.shape[1]
    H2 = W2.shape[1]

    NP = 10240
    n_pad_rows = NP - N

    src = edge_index[0].astype(jnp.int32)
    dst = edge_index[1].astype(jnp.int32)
    edges3d, Epad = _pad_edges(src, dst, N, n_pad_rows)
    n_chunks_per_tile = Epad // (NW * CHUNK)

    z1 = jnp.zeros((NP, H1), jnp.float32)
    z2 = jnp.zeros((NP, H2), jnp.float32)

    hw1 = _tc_matmul_bias(features, W1, b1.reshape(1, H1), block_rows=400)
    p10, p11 = _sc_segment_sum(hw1, edges3d, z1, NP, n_chunks_per_tile,
                               tc_tiling=True)

    hw2 = _tc_relu_add_matmul_bias(p10, p11, W2, b2.reshape(1, H2))
    p20, p21 = _sc_segment_sum(hw2, edges3d, z2, NP, n_chunks_per_tile,
                               tc_tiling=False, stage_src=True)

    pa = jnp.concatenate(
        [positive_edge_pairs[0], negative_edge_pairs[0]]
    ).astype(jnp.int32)
    pb = jnp.concatenate(
        [positive_edge_pairs[1], negative_edge_pairs[1]]
    ).astype(jnp.int32)
    P = pa.shape[0]
    round_sz = NW * CHUNK
    Ppad = ((P + round_sz - 1) // round_sz) * round_sz
    padp = Ppad - P
    if padp:
        ar = jnp.arange(padp, dtype=jnp.int32)
        pa = jnp.concatenate([pa, (ar * 131) % N])
        pb = jnp.concatenate([pb, (ar * 173) % N])
    pairs3d = jnp.stack([pa.reshape(-1, CHUNK), pb.reshape(-1, CHUNK)], axis=1)

    result = _sc_decoder(p20, p21, pairs3d, Ppad // (NW * CHUNK))
    return result[:P]


# speedup vs baseline: 1.2670x; 1.0440x over previous
import functools

import jax
import jax.numpy as jnp
from jax import lax
from jax.experimental import pallas as pl
from jax.experimental.pallas import tpu as pltpu
from jax.experimental.pallas import tpu_sc as plsc

N_TILES = 16
N_CORES = 2
NW = N_TILES * N_CORES
CHUNK = 128
GRP = 8
STEPS = 2 * GRP


def _tc_matmul_bias(x, w, b, block_rows):
    M, K = x.shape
    D = w.shape[1]

    def body(x_ref, w_ref, b_ref, o_ref):
        o_ref[...] = (
            jnp.dot(x_ref[...], w_ref[...], preferred_element_type=jnp.float32)
            + b_ref[...]
        )

    return pl.pallas_call(
        body,
        out_shape=jax.ShapeDtypeStruct((M, D), jnp.float32),
        grid=(M // block_rows,),
        in_specs=[
            pl.BlockSpec((block_rows, K), lambda i: (i, 0)),
            pl.BlockSpec((K, D), lambda i: (0, 0)),
            pl.BlockSpec((1, D), lambda i: (0, 0)),
        ],
        out_specs=pl.BlockSpec((block_rows, D), lambda i: (i, 0)),
    )(x, w, b)


def _tc_relu_add_matmul_bias(p0, p1, w, b, block_rows=512):
    M, K = p0.shape
    D = w.shape[1]

    def body(p0_ref, p1_ref, w_ref, b_ref, o_ref):
        h = jnp.maximum(p0_ref[...] + p1_ref[...], 0.0)
        o_ref[...] = (
            jnp.dot(h, w_ref[...], preferred_element_type=jnp.float32) + b_ref[...]
        )

    return pl.pallas_call(
        body,
        out_shape=jax.ShapeDtypeStruct((M, D), jnp.float32),
        grid=(M // block_rows,),
        in_specs=[
            pl.BlockSpec((block_rows, K), lambda i: (i, 0)),
            pl.BlockSpec((block_rows, K), lambda i: (i, 0)),
            pl.BlockSpec((K, D), lambda i: (0, 0)),
            pl.BlockSpec((1, D), lambda i: (0, 0)),
        ],
        out_specs=pl.BlockSpec((block_rows, D), lambda i: (i, 0)),
    )(p0, p1, w, b)


def _tc_add(p0, p1, block_rows=512):
    M, D = p0.shape

    def body(a_ref, b_ref, o_ref):
        o_ref[...] = a_ref[...] + b_ref[...]

    return pl.pallas_call(
        body,
        out_shape=jax.ShapeDtypeStruct((M, D), jnp.float32),
        grid=(M // block_rows,),
        in_specs=[
            pl.BlockSpec((block_rows, D), lambda i: (i, 0)),
            pl.BlockSpec((block_rows, D), lambda i: (i, 0)),
        ],
        out_specs=pl.BlockSpec((block_rows, D), lambda i: (i, 0)),
    )(p0, p1)


def _sc_segment_sum(hw, edges3d, zeros_np, NP, n_chunks_per_tile,
                    tc_tiling=True, stage_src=False):
    NH, D = hw.shape
    NCH = n_chunks_per_tile
    T = NCH // STEPS
    assert NCH == T * STEPS
    assert not stage_src or NH == NP
    rows_per_tile = NP // N_TILES
    mesh = plsc.VectorSubcoreMesh(core_axis_name="c", subcore_axis_name="s")

    scratch = [
        pltpu.VMEM((GRP, 2, CHUNK), jnp.int32),
        pltpu.VMEM((GRP, 2, CHUNK), jnp.int32),
        pltpu.VMEM((CHUNK, D), jnp.float32),
        pltpu.VMEM((CHUNK, D), jnp.float32),
        pltpu.VMEM_SHARED((NP, D), jnp.float32),
        pltpu.SemaphoreType.DMA,
        pltpu.SemaphoreType.DMA,
        pltpu.SemaphoreType.DMA,
        pltpu.SemaphoreType.DMA,
    ]
    if stage_src:
        scratch.append(pltpu.VMEM_SHARED((NP, D), jnp.float32))

    @functools.partial(
        pl.kernel,
        out_type=[
            jax.ShapeDtypeStruct((NP, D), jnp.float32),
            jax.ShapeDtypeStruct((NP, D), jnp.float32),
        ],
        mesh=mesh,
        scratch_types=scratch,
        compiler_params=pltpu.CompilerParams(use_tc_tiling_on_sc=tc_tiling),
    )
    def k(hw_hbm, edges_hbm, zeros_hbm, out0_hbm, out1_hbm, idxA, idxB,
          rows0, rows1, acc, gsem0, gsem1, isemA, isemB, *maybe_stage):
        cid = lax.axis_index("c")
        sid = lax.axis_index("s")
        wid = sid * N_CORES + cid
        r0 = sid * rows_per_tile
        base = wid * NCH
        rows = (rows0, rows1)
        gsems = (gsem0, gsem1)

        pltpu.sync_copy(
            zeros_hbm.at[pl.ds(r0, rows_per_tile)],
            acc.at[pl.ds(r0, rows_per_tile)],
        )
        if stage_src:
            src_ref = maybe_stage[0]
            pltpu.sync_copy(
                hw_hbm.at[pl.ds(r0, rows_per_tile)],
                src_ref.at[pl.ds(r0, rows_per_tile)],
            )
        else:
            src_ref = hw_hbm
        plsc.subcore_barrier()

        def start_g(idx, row, p):
            pltpu.async_copy(src_ref.at[idx.at[row, 0]], rows[p], gsems[p])

        def wait_g(idx, row, p):
            pltpu.make_async_copy(
                src_ref.at[idx.at[row, 0]], rows[p], gsems[p]
            ).wait()

        def scat(idx, row, p):
            pltpu.sync_copy(rows[p], acc.at[idx.at[row, 1]], add=True)

        pltpu.sync_copy(edges_hbm.at[pl.ds(base, GRP)], idxA)
        start_g(idxA, 0, 0)

        def outer(t, carry):
            for u in range(STEPS):
                p = u % 2
                q = 1 - p
                idx_cur = idxA if u < GRP else idxB
                row_cur = u % GRP
                if u == 0:
                    pltpu.async_copy(
                        edges_hbm.at[pl.ds(base + t * STEPS + GRP, GRP)],
                        idxB, isemB,
                    )
                if u == GRP:
                    @pl.when(t + 1 < T)
                    def _():
                        pltpu.async_copy(
                            edges_hbm.at[pl.ds(base + (t + 1) * STEPS, GRP)],
                            idxA, isemA,
                        )
                if u < GRP - 1:
                    start_g(idxA, u + 1, q)
                elif u == GRP - 1:
                    pltpu.make_async_copy(
                        edges_hbm.at[pl.ds(base + t * STEPS + GRP, GRP)],
                        idxB, isemB,
                    ).wait()
                    start_g(idxB, 0, q)
                elif u < STEPS - 1:
                    start_g(idxB, u - GRP + 1, q)
                else:
                    @pl.when(t + 1 < T)
                    def _():
                        pltpu.make_async_copy(
                            edges_hbm.at[pl.ds(base + (t + 1) * STEPS, GRP)],
                            idxA, isemA,
                        ).wait()
                        start_g(idxA, 0, q)
                wait_g(idx_cur, row_cur, p)
                scat(idx_cur, row_cur, p)
            return carry

        lax.fori_loop(0, T, outer, 0)
        plsc.subcore_barrier()

        @pl.when(cid == 0)
        def _():
            pltpu.sync_copy(
                acc.at[pl.ds(r0, rows_per_tile)],
                out0_hbm.at[pl.ds(r0, rows_per_tile)],
            )

        @pl.when(cid == 1)
        def _():
            pltpu.sync_copy(
                acc.at[pl.ds(r0, rows_per_tile)],
                out1_hbm.at[pl.ds(r0, rows_per_tile)],
            )

    return k(hw, edges3d, zeros_np)


def _sc_decoder(q0, q1, pairs3d, n_chunks_per_tile):
    NP, D = q0.shape
    CH = pairs3d.shape[0]
    mesh = plsc.VectorSubcoreMesh(core_axis_name="c", subcore_axis_name="s")

    NCH = n_chunks_per_tile
    assert NCH % 2 == 0
    rows_per_tile = NP // N_TILES
    n_stage_chunks = rows_per_tile // CHUNK

    @functools.partial(
        pl.kernel,
        out_type=jax.ShapeDtypeStruct((CH * CHUNK,), jnp.float32),
        mesh=mesh,
        scratch_types=[
            pltpu.VMEM((NCH, 2, CHUNK), jnp.int32),
            pltpu.VMEM((2, CHUNK, D), jnp.float32),
            pltpu.VMEM((2, CHUNK, D), jnp.float32),
            pltpu.VMEM((CHUNK,), jnp.float32),
            pltpu.VMEM((CHUNK, D), jnp.float32),
            pltpu.VMEM((n_stage_chunks, 1, CHUNK), jnp.int32),
            pltpu.VMEM_SHARED((NP, D), jnp.float32),
            pltpu.SemaphoreType.DMA,
            pltpu.SemaphoreType.DMA,
        ],
        compiler_params=pltpu.CompilerParams(
            use_tc_tiling_on_sc=False, needs_layout_passes=False
        ),
    )
    def k(q0_hbm, q1_hbm, pairs_hbm, out_hbm, idx_all, ra, rb,
          res_v, qbuf, iidx, h2s, sem0, sem1):
        cid = lax.axis_index("c")
        sid = lax.axis_index("s")
        wid = sid * N_CORES + cid
        base = wid * NCH
        r0 = sid * rows_per_tile

        lane = lax.iota(jnp.int32, 16)
        pltpu.sync_copy(pairs_hbm.at[pl.ds(base, NCH)], idx_all)

        for c in range(n_stage_chunks):
            for s in range(CHUNK // 16):
                iidx[c, 0, pl.ds(s * 16, 16)] = r0 + c * CHUNK + s * 16 + lane
        pltpu.sync_copy(
            q0_hbm.at[pl.ds(r0, rows_per_tile)],
            h2s.at[pl.ds(r0, rows_per_tile)],
        )
        for c in range(n_stage_chunks):
            pltpu.sync_copy(q1_hbm.at[pl.ds(r0 + c * CHUNK, CHUNK)], qbuf)
            pltpu.sync_copy(qbuf, h2s.at[iidx.at[c, 0]], add=True)
        plsc.subcore_barrier()

        def start_gather(j, buf, sem):
            pltpu.async_copy(h2s.at[idx_all.at[j, 0]], ra.at[buf], sem)
            pltpu.async_copy(h2s.at[idx_all.at[j, 1]], rb.at[buf], sem)

        def drain_gather(j, buf, sem):
            pltpu.make_async_copy(
                h2s.at[idx_all.at[j, 0]], ra.at[buf], sem
            ).wait()
            pltpu.make_async_copy(
                h2s.at[idx_all.at[j, 1]], rb.at[buf], sem
            ).wait()

        def compute(j, buf):
            def group_body(g, c2):
                res = jnp.zeros((16,), jnp.float32)
                for p2 in range(16):
                    p = g * 16 + p2
                    acc = ra[buf, p, pl.ds(0, 16)] * rb[buf, p, pl.ds(0, 16)]
                    acc = acc + ra[buf, p, pl.ds(16, 16)] * rb[buf, p, pl.ds(16, 16)]
                    acc = acc + ra[buf, p, pl.ds(32, 16)] * rb[buf, p, pl.ds(32, 16)]
                    acc = acc + ra[buf, p, pl.ds(48, 16)] * rb[buf, p, pl.ds(48, 16)]
                    res = jnp.where(lane == p2, jnp.sum(acc), res)
                res_v[pl.ds(g * 16, 16)] = res
                return c2

            lax.fori_loop(0, CHUNK // 16, group_body, 0)
            pltpu.sync_copy(
                res_v, out_hbm.at[pl.ds((base + j) * CHUNK, CHUNK)]
            )

        start_gather(0, 0, sem0)

        def body(k2, carry):
            j = 2 * k2
            start_gather(j + 1, 1, sem1)
            drain_gather(j, 0, sem0)
            compute(j, 0)

            @pl.when(k2 + 1 < NCH // 2)
            def _():
                start_gather(j + 2, 0, sem0)

            drain_gather(j + 1, 1, sem1)
            compute(j + 1, 1)
            return carry

        lax.fori_loop(0, NCH // 2, body, 0)

    return k(q0, q1, pairs3d)


def _pad_edges(src, dst, n_real_rows, n_pad_rows):
    E = src.shape[0]
    round_sz = STEPS * NW * CHUNK
    Epad = ((E + round_sz - 1) // round_sz) * round_sz
    pad = Epad - E
    if pad:
        ar = jnp.arange(pad, dtype=jnp.int32)
        src = jnp.concatenate([src, (ar * 97) % n_real_rows])
        dst = jnp.concatenate([dst, n_real_rows + ar % n_pad_rows])
    edges3d = jnp.stack(
        [src.reshape(-1, CHUNK), dst.reshape(-1, CHUNK)], axis=1
    )
    return edges3d, Epad


def kernel(features, edge_index, positive_edge_pairs, negative_edge_pairs,
           W1, b1, W2, b2):
    N, D_IN = features.shape
    H1 = W1.shape[1]
    H2 = W2.shape[1]

    NP = 10240
    n_pad_rows = NP - N

    src = edge_index[0].astype(jnp.int32)
    dst = edge_index[1].astype(jnp.int32)
    edges3d, Epad = _pad_edges(src, dst, N, n_pad_rows)
    n_chunks_per_tile = Epad // (NW * CHUNK)

    z1 = jnp.zeros((NP, H1), jnp.float32)
    z2 = jnp.zeros((NP, H2), jnp.float32)

    hw1 = _tc_matmul_bias(features, W1, b1.reshape(1, H1), block_rows=400)
    p10, p11 = _sc_segment_sum(hw1, edges3d, z1, NP, n_chunks_per_tile,
                               tc_tiling=True)

    hw2 = _tc_relu_add_matmul_bias(p10, p11, W2, b2.reshape(1, H2))
    p20, p21 = _sc_segment_sum(hw2, edges3d, z2, NP, n_chunks_per_tile,
                               tc_tiling=False)

    pa = jnp.concatenate(
        [positive_edge_pairs[0], negative_edge_pairs[0]]
    ).astype(jnp.int32)
    pb = jnp.concatenate(
        [positive_edge_pairs[1], negative_edge_pairs[1]]
    ).astype(jnp.int32)
    P = pa.shape[0]
    round_sz = NW * CHUNK
    Ppad = ((P + round_sz - 1) // round_sz) * round_sz
    padp = Ppad - P
    if padp:
        ar = jnp.arange(padp, dtype=jnp.int32)
        pa = jnp.concatenate([pa, (ar * 131) % N])
        pb = jnp.concatenate([pb, (ar * 173) % N])
    pairs3d = jnp.stack([pa.reshape(-1, CHUNK), pb.reshape(-1, CHUNK)], axis=1)

    result = _sc_decoder(p20, p21, pairs3d, Ppad // (NW * CHUNK))
    return result[:P]
